# Initial kernel scaffold; baseline (speedup 1.0000x reference)
import jax
import jax.numpy as jnp
from jax.experimental import pallas as pl


def kernel(x, pos_emb):
    raise NotImplementedError("write your pallas kernel here")


# speedup vs baseline: 1.6676x; 1.6676x over previous
import jax
import jax.numpy as jnp
from jax.experimental import pallas as pl
from jax.experimental.pallas import tpu as pltpu

_ROWS = 1024


def _add_body(x_ref, pe_ref, out_ref):
    out_ref[...] = x_ref[...] + pe_ref[...][None]


def kernel(x, pos_emb):
    B, T, C = x.shape
    grid = (T // _ROWS, B)
    return pl.pallas_call(
        _add_body,
        grid=grid,
        in_specs=[
            pl.BlockSpec((1, _ROWS, C), lambda t, b: (b, t, 0)),
            pl.BlockSpec((_ROWS, C), lambda t, b: (t, 0)),
        ],
        out_specs=pl.BlockSpec((1, _ROWS, C), lambda t, b: (b, t, 0)),
        out_shape=jax.ShapeDtypeStruct((B, T, C), x.dtype),
    )(x, pos_emb)


# speedup vs baseline: 1.7356x; 1.0408x over previous
import jax
import jax.numpy as jnp
from jax.experimental import pallas as pl
from jax.experimental.pallas import tpu as pltpu

_ROWS = 2048


def _add_body(x_ref, pe_ref, out_ref):
    out_ref[...] = x_ref[...] + pe_ref[...][None]


def kernel(x, pos_emb):
    B, T, C = x.shape
    grid = (T // _ROWS, B)
    return pl.pallas_call(
        _add_body,
        grid=grid,
        in_specs=[
            pl.BlockSpec((1, _ROWS, C), lambda t, b: (b, t, 0)),
            pl.BlockSpec((_ROWS, C), lambda t, b: (t, 0)),
        ],
        out_specs=pl.BlockSpec((1, _ROWS, C), lambda t, b: (b, t, 0)),
        out_shape=jax.ShapeDtypeStruct((B, T, C), x.dtype),
    )(x, pos_emb)
